# baseline (speedup 1.0000x reference)
import jax
import jax.numpy as jnp
from jax.experimental import pallas as pl
from jax.experimental.pallas import tpu as pltpu

H_BLK = 512
N_CORES = 2


def kernel(x, fc1_w, fc1_b, fc2_w, fc2_b):
    E, CAP, D = x.shape
    H = fc1_w.shape[1]
    n_h = H // H_BLK
    epc = E // N_CORES
    S = epc * n_h + 1
    b1r = fc1_b.reshape(E, 1, H)
    b2r = fc2_b.reshape(E, 1, D)

    def body(x_ref, w1_ref, b1_ref, w2_ref, b2_ref, o_ref, xb_ref, y_ref):
        s = pl.program_id(1)

        @pl.when(jax.lax.rem(s, n_h) == 0)
        def _():
            xb_ref[...] = x_ref[0].astype(jnp.bfloat16)

        @pl.when(jax.lax.rem(s, n_h) == 1)
        def _():
            o_ref[0] = jnp.broadcast_to(b2_ref[0], (CAP, D))

        def both(cur, prv):
            w2 = w2_ref[0].astype(jnp.bfloat16)
            acc = jax.lax.dot_general(
                y_ref[prv], w2, (((1,), (0,)), ((), ())),
                preferred_element_type=jnp.float32)
            o_ref[0] += acc
            w1 = w1_ref[0].astype(jnp.bfloat16)
            y = jax.lax.dot_general(
                xb_ref[...], w1, (((1,), (1,)), ((), ())),
                preferred_element_type=jnp.float32)
            y_ref[cur] = jnp.maximum(y + b1_ref[0], 0.0).astype(jnp.bfloat16)

        @pl.when(jax.lax.rem(s, 2) == 0)
        def _():
            both(0, 1)

        @pl.when(jax.lax.rem(s, 2) == 1)
        def _():
            both(1, 0)

    def cur_e(c, s):
        return c * epc + jnp.minimum(s // n_h, epc - 1)

    def cur_h(c, s):
        return jnp.where(s // n_h > epc - 1, n_h - 1, jax.lax.rem(s, n_h))

    def prv_e(c, s):
        t = jnp.maximum(s - 1, 0)
        return c * epc + t // n_h

    def prv_h(c, s):
        t = jnp.maximum(s - 1, 0)
        return jax.lax.rem(t, n_h)

    return pl.pallas_call(
        body,
        grid=(N_CORES, S),
        in_specs=[
            pl.BlockSpec((1, CAP, D), lambda c, s: (cur_e(c, s), 0, 0)),
            pl.BlockSpec((1, H_BLK, D), lambda c, s: (cur_e(c, s), cur_h(c, s), 0)),
            pl.BlockSpec((1, 1, H_BLK), lambda c, s: (cur_e(c, s), 0, cur_h(c, s))),
            pl.BlockSpec((1, H_BLK, D), lambda c, s: (prv_e(c, s), prv_h(c, s), 0)),
            pl.BlockSpec((1, 1, D), lambda c, s: (prv_e(c, s), 0, 0)),
        ],
        out_specs=pl.BlockSpec((1, CAP, D), lambda c, s: (prv_e(c, s), 0, 0)),
        out_shape=jax.ShapeDtypeStruct((E, CAP, D), jnp.float32),
        scratch_shapes=[
            pltpu.VMEM((CAP, D), jnp.bfloat16),
            pltpu.VMEM((2, CAP, H_BLK), jnp.bfloat16),
        ],
        compiler_params=pltpu.CompilerParams(
            dimension_semantics=("parallel", "arbitrary"),
            vmem_limit_bytes=100 * 1024 * 1024,
        ),
        name="fused_expert_ffn",
    )(x, fc1_w, b1r, fc2_w, b2r)


# speedup vs baseline: 1.0227x; 1.0227x over previous
import jax
import jax.numpy as jnp
from jax.experimental import pallas as pl
from jax.experimental.pallas import tpu as pltpu

H_BLK = 1024
N_CORES = 2


def kernel(x, fc1_w, fc1_b, fc2_w, fc2_b):
    E, CAP, D = x.shape
    H = fc1_w.shape[1]
    n_h = H // H_BLK
    epc = E // N_CORES
    S = epc * n_h + 1
    b1r = fc1_b.reshape(E, 1, H)
    b2r = fc2_b.reshape(E, 1, D)

    def body(x_ref, w1_ref, b1_ref, w2_ref, b2_ref, o_ref, y_ref):
        s = pl.program_id(1)

        @pl.when(jax.lax.rem(s, n_h) == 1)
        def _():
            o_ref[0] = jnp.broadcast_to(b2_ref[0], (CAP, D))

        def both(cur, prv):
            w2 = w2_ref[0].astype(jnp.bfloat16)
            acc = jax.lax.dot_general(
                y_ref[prv], w2, (((1,), (0,)), ((), ())),
                preferred_element_type=jnp.float32)
            o_ref[0] += acc
            w1 = w1_ref[0].astype(jnp.bfloat16)
            y = jax.lax.dot_general(
                x_ref[0].astype(jnp.bfloat16), w1, (((1,), (1,)), ((), ())),
                preferred_element_type=jnp.float32)
            y_ref[cur] = jnp.maximum(y + b1_ref[0], 0.0).astype(jnp.bfloat16)

        @pl.when(jax.lax.rem(s, 2) == 0)
        def _():
            both(0, 1)

        @pl.when(jax.lax.rem(s, 2) == 1)
        def _():
            both(1, 0)

    def cur_e(c, s):
        return c * epc + jnp.minimum(s // n_h, epc - 1)

    def cur_h(c, s):
        return jnp.where(s // n_h > epc - 1, n_h - 1, jax.lax.rem(s, n_h))

    def prv_e(c, s):
        t = jnp.maximum(s - 1, 0)
        return c * epc + t // n_h

    def prv_h(c, s):
        t = jnp.maximum(s - 1, 0)
        return jax.lax.rem(t, n_h)

    return pl.pallas_call(
        body,
        grid=(N_CORES, S),
        in_specs=[
            pl.BlockSpec((1, CAP, D), lambda c, s: (cur_e(c, s), 0, 0)),
            pl.BlockSpec((1, H_BLK, D), lambda c, s: (cur_e(c, s), cur_h(c, s), 0)),
            pl.BlockSpec((1, 1, H_BLK), lambda c, s: (cur_e(c, s), 0, cur_h(c, s))),
            pl.BlockSpec((1, H_BLK, D), lambda c, s: (prv_e(c, s), prv_h(c, s), 0)),
            pl.BlockSpec((1, 1, D), lambda c, s: (prv_e(c, s), 0, 0)),
        ],
        out_specs=pl.BlockSpec((1, CAP, D), lambda c, s: (prv_e(c, s), 0, 0)),
        out_shape=jax.ShapeDtypeStruct((E, CAP, D), jnp.float32),
        scratch_shapes=[
            pltpu.VMEM((2, CAP, H_BLK), jnp.bfloat16),
        ],
        compiler_params=pltpu.CompilerParams(
            dimension_semantics=("parallel", "arbitrary"),
            vmem_limit_bytes=66584576,
        ),
        name="fused_expert_ffn",
    )(x, fc1_w, b1r, fc2_w, b2r)


# speedup vs baseline: 1.0358x; 1.0128x over previous
import jax
import jax.numpy as jnp
from jax.experimental import pallas as pl
from jax.experimental.pallas import tpu as pltpu

H_BLK = 1024


def kernel(x, fc1_w, fc1_b, fc2_w, fc2_b):
    E, CAP, D = x.shape
    H = fc1_w.shape[1]
    n_h = H // H_BLK
    P = CAP // n_h
    T = E * n_h
    S = n_h + T + 1
    b1r = fc1_b.reshape(E, 1, H)
    b2r = fc2_b.reshape(E, 1, D)

    def body(xp_ref, w1_ref, b1_ref, w2_ref, b2_ref, o_ref, xb_ref, y_ref):
        s = pl.program_id(0)
        t = jnp.clip(s - n_h, 0, T - 1)
        tp = jnp.clip(s - n_h - 1, 0, T - 1)

        @pl.when(jnp.logical_and(s >= n_h, jax.lax.rem(tp, n_h) == 0))
        def _():
            o_ref[0] = jnp.broadcast_to(b2_ref[0], (CAP, D))

        def both(cur, prv):
            w2 = w2_ref[0].astype(jnp.bfloat16)
            acc = jax.lax.dot_general(
                y_ref[prv], w2, (((1,), (0,)), ((), ())),
                preferred_element_type=jnp.float32)
            o_ref[0] += acc
            w1 = w1_ref[0].astype(jnp.bfloat16)
            y = jax.lax.dot_general(
                xb_ref[jax.lax.rem(t // n_h, 2)], w1,
                (((1,), (1,)), ((), ())),
                preferred_element_type=jnp.float32)
            y_ref[cur] = jnp.maximum(y + b1_ref[0], 0.0).astype(jnp.bfloat16)

        @pl.when(jnp.logical_and(s >= n_h, jax.lax.rem(s, 2) == 0))
        def _():
            both(0, 1)

        @pl.when(jnp.logical_and(s >= n_h, jax.lax.rem(s, 2) == 1))
        def _():
            both(1, 0)

        @pl.when(s < T)
        def _():
            pe = s // n_h
            pq = jax.lax.rem(s, n_h)
            xb_ref[jax.lax.rem(pe, 2), pl.ds(pq * P, P), :] = (
                xp_ref[0].astype(jnp.bfloat16))

    def pc(s):
        p = jnp.clip(s, 0, T - 1)
        return p // n_h, jax.lax.rem(p, n_h)

    def cur(s):
        t = jnp.clip(s - n_h, 0, T - 1)
        return t // n_h, jax.lax.rem(t, n_h)

    def prv(s):
        t = jnp.clip(s - n_h - 1, 0, T - 1)
        return t // n_h, jax.lax.rem(t, n_h)

    return pl.pallas_call(
        body,
        grid=(S,),
        in_specs=[
            pl.BlockSpec((1, P, D), lambda s: (pc(s)[0], pc(s)[1], 0)),
            pl.BlockSpec((1, H_BLK, D), lambda s: (cur(s)[0], cur(s)[1], 0)),
            pl.BlockSpec((1, 1, H_BLK), lambda s: (cur(s)[0], 0, cur(s)[1])),
            pl.BlockSpec((1, H_BLK, D), lambda s: (prv(s)[0], prv(s)[1], 0)),
            pl.BlockSpec((1, 1, D), lambda s: (prv(s)[0], 0, 0)),
        ],
        out_specs=pl.BlockSpec((1, CAP, D), lambda s: (prv(s)[0], 0, 0)),
        out_shape=jax.ShapeDtypeStruct((E, CAP, D), jnp.float32),
        scratch_shapes=[
            pltpu.VMEM((2, CAP, D), jnp.bfloat16),
            pltpu.VMEM((2, CAP, H_BLK), jnp.bfloat16),
        ],
        compiler_params=pltpu.CompilerParams(
            dimension_semantics=("arbitrary",),
            vmem_limit_bytes=66584576,
        ),
        name="fused_expert_ffn",
    )(x, fc1_w, b1r, fc2_w, b2r)


# speedup vs baseline: 1.0517x; 1.0154x over previous
import jax
import jax.numpy as jnp
from jax.experimental import pallas as pl
from jax.experimental.pallas import tpu as pltpu

H_BLK = 1024


def kernel(x, fc1_w, fc1_b, fc2_w, fc2_b):
    E, CAP, D = x.shape
    H = fc1_w.shape[1]
    n_h = H // H_BLK
    P = CAP // n_h
    T = E * n_h
    S = n_h + T + 1
    b1r = fc1_b.reshape(E, 1, H)
    b2r = fc2_b.reshape(E, 1, D)

    def body(xp_ref, w1_ref, b1_ref, w2_ref, b2_ref, o_ref, xb_ref, y_ref):
        s = pl.program_id(0)
        t = jnp.clip(s - n_h, 0, T - 1)
        tp = jnp.clip(s - n_h - 1, 0, T - 1)

        @pl.when(jnp.logical_and(s >= n_h, jax.lax.rem(tp, n_h) == 0))
        def _():
            o_ref[0] = jnp.broadcast_to(b2_ref[tp // n_h], (CAP, D))

        def both(cur, prv):
            w2 = w2_ref[0].astype(jnp.bfloat16)
            acc = jax.lax.dot_general(
                y_ref[prv], w2, (((1,), (0,)), ((), ())),
                preferred_element_type=jnp.float32)
            o_ref[0] += acc
            w1 = w1_ref[0].astype(jnp.bfloat16)
            y = jax.lax.dot_general(
                xb_ref[jax.lax.rem(t // n_h, 2)], w1,
                (((1,), (1,)), ((), ())),
                preferred_element_type=jnp.float32)
            b1 = b1_ref[t // n_h, :, pl.ds(jax.lax.rem(t, n_h) * H_BLK, H_BLK)]
            y_ref[cur] = jnp.maximum(y + b1, 0.0).astype(jnp.bfloat16)

        steady = jnp.logical_and(s >= n_h, s < S - 1)

        @pl.when(jnp.logical_and(steady, jax.lax.rem(s, 2) == 0))
        def _():
            both(0, 1)

        @pl.when(jnp.logical_and(steady, jax.lax.rem(s, 2) == 1))
        def _():
            both(1, 0)

        @pl.when(s == S - 1)
        def _():
            w2 = w2_ref[0].astype(jnp.bfloat16)
            acc = jax.lax.dot_general(
                y_ref[1], w2, (((1,), (0,)), ((), ())),
                preferred_element_type=jnp.float32)
            o_ref[0] += acc

        @pl.when(s < T)
        def _():
            pe = s // n_h
            pq = jax.lax.rem(s, n_h)
            xb_ref[jax.lax.rem(pe, 2), pl.ds(pq * P, P), :] = (
                xp_ref[0].astype(jnp.bfloat16))

    def pc(s):
        p = jnp.clip(s, 0, T - 1)
        return p // n_h, jax.lax.rem(p, n_h)

    def cur(s):
        t = jnp.clip(s - n_h, 0, T - 1)
        return t // n_h, jax.lax.rem(t, n_h)

    def prv(s):
        t = jnp.clip(s - n_h - 1, 0, T - 1)
        return t // n_h, jax.lax.rem(t, n_h)

    return pl.pallas_call(
        body,
        grid=(S,),
        in_specs=[
            pl.BlockSpec((1, P, D), lambda s: (pc(s)[0], pc(s)[1], 0)),
            pl.BlockSpec((1, H_BLK, D), lambda s: (cur(s)[0], cur(s)[1], 0)),
            pl.BlockSpec((E, 1, H), lambda s: (0, 0, 0)),
            pl.BlockSpec((1, H_BLK, D), lambda s: (prv(s)[0], prv(s)[1], 0)),
            pl.BlockSpec((E, 1, D), lambda s: (0, 0, 0)),
        ],
        out_specs=pl.BlockSpec((1, CAP, D), lambda s: (prv(s)[0], 0, 0)),
        out_shape=jax.ShapeDtypeStruct((E, CAP, D), jnp.float32),
        scratch_shapes=[
            pltpu.VMEM((2, CAP, D), jnp.bfloat16),
            pltpu.VMEM((2, CAP, H_BLK), jnp.bfloat16),
        ],
        compiler_params=pltpu.CompilerParams(
            dimension_semantics=("arbitrary",),
            vmem_limit_bytes=66584576,
        ),
        name="fused_expert_ffn",
    )(x, fc1_w, b1r, fc2_w, b2r)
